# baseline (speedup 1.0000x reference)
import functools

import jax
import jax.numpy as jnp
from jax import lax
from jax.experimental import pallas as pl
from jax.experimental.pallas import tpu as pltpu
from jax.experimental.pallas import tpu_sc as plsc

D = 768
L = 16
NJ = D // L
C = 8
NSEQ = 4
NTOK = NSEQ * C
EPS = 1e-12


def _rsqrt_vec(x):
    i = plsc.bitcast(x, jnp.int32)
    i = jnp.int32(0x5F3759DF) - (i >> 1)
    y = plsc.bitcast(i, jnp.float32)
    for _ in range(3):
        y = y * (1.5 - 0.5 * x * y * y)
    return y


@functools.partial(jax.jit, static_argnames=("n_tokens", "seq_len"))
def _embed_ln(ids, tts, word_emb, pos_emb, type_emb, gamma, beta, *,
              n_tokens, seq_len):
    info = plsc.get_sparse_core_info()
    nw = info.num_cores * info.num_subcores
    n_seqs = n_tokens // seq_len
    assert n_seqs == nw * NSEQ
    n_sc = seq_len // C
    n_type = type_emb.shape[0]
    mesh = plsc.VectorSubcoreMesh(core_axis_name="c", subcore_axis_name="s")

    scratch = (
        [pltpu.VMEM((2, NSEQ, C, D), jnp.float32),
         pltpu.VMEM((2, C, D), jnp.float32),
         pltpu.VMEM((2, NSEQ, C), jnp.int32),
         pltpu.VMEM((2, NSEQ, C), jnp.int32),
         pltpu.VMEM((D,), jnp.float32),
         pltpu.VMEM((D,), jnp.float32),
         pltpu.VMEM((n_type, D), jnp.float32),
         pltpu.VMEM((NTOK, L), jnp.float32),
         pltpu.VMEM((NTOK, L), jnp.float32),
         pltpu.VMEM((NTOK,), jnp.float32),
         pltpu.VMEM((NTOK,), jnp.float32),
         pltpu.VMEM((NSEQ, C, D), jnp.float32)]
        + [pltpu.SemaphoreType.DMA for _ in range(8)]
    )

    @functools.partial(
        pl.kernel,
        out_type=jax.ShapeDtypeStruct((n_seqs, seq_len, D), jnp.float32),
        mesh=mesh,
        scratch_types=scratch,
        compiler_params=pltpu.CompilerParams(needs_layout_passes=False),
    )
    def k(ids_hbm, tts_hbm, word_hbm, pos_hbm, type_hbm, gamma_hbm, beta_hbm,
          out_hbm, *sc):
        rows4 = sc[0]
        posb = sc[1]
        ids4 = sc[2]
        tt4 = sc[3]
        gamma_v, beta_v, type_v = sc[4], sc[5], sc[6]
        s1b, s2b, meanb, invb = sc[7], sc[8], sc[9], sc[10]
        ebuf = sc[11]
        sem_ids = sc[12:14]
        sem_pos = sc[14:16]
        sem_w = sc[16:18]
        sem_o = sc[18:20]

        cid = lax.axis_index("c")
        sid = lax.axis_index("s")
        wid = sid * info.num_cores + cid
        seq0 = wid * NSEQ

        pltpu.sync_copy(gamma_hbm, gamma_v)
        pltpu.sync_copy(beta_hbm, beta_v)
        pltpu.sync_copy(type_hbm, type_v)

        def issue_ids(sck, slot):
            for b in range(NSEQ):
                pltpu.async_copy(
                    ids_hbm.at[seq0 + b, pl.ds(sck * C, C)],
                    ids4.at[slot, b], sem_ids[slot])
                pltpu.async_copy(
                    tts_hbm.at[seq0 + b, pl.ds(sck * C, C)],
                    tt4.at[slot, b], sem_ids[slot])

        def wait_ids(slot):
            for b in range(NSEQ):
                pltpu.make_async_copy(ids_hbm.at[0, pl.ds(0, C)],
                                      ids4.at[slot, b], sem_ids[slot]).wait()
                pltpu.make_async_copy(tts_hbm.at[0, pl.ds(0, C)],
                                      tt4.at[slot, b], sem_ids[slot]).wait()

        def issue_pos(sck, slot):
            pltpu.async_copy(pos_hbm.at[pl.ds(sck * C, C)],
                             posb.at[slot], sem_pos[slot])

        def wait_pos(slot):
            pltpu.make_async_copy(pos_hbm.at[pl.ds(0, C)], posb.at[slot],
                                  sem_pos[slot]).wait()

        def issue_word(slot):
            for b in range(NSEQ):
                pltpu.async_copy(word_hbm.at[ids4.at[slot, b]],
                                 rows4.at[slot, b], sem_w[slot])

        def wait_word(slot):
            for b in range(NSEQ):
                pltpu.make_async_copy(word_hbm.at[ids4.at[slot, b]],
                                     rows4.at[slot, b], sem_w[slot]).wait()

        def issue_out(sck, slot):
            pltpu.async_copy(
                rows4.at[slot],
                out_hbm.at[pl.ds(seq0, NSEQ), pl.ds(sck * C, C)],
                sem_o[slot])

        def wait_out(slot):
            pltpu.make_async_copy(rows4.at[slot],
                                  out_hbm.at[pl.ds(0, NSEQ), pl.ds(0, C)],
                                  sem_o[slot]).wait()

        iota = lax.iota(jnp.int32, L)
        zeros = jnp.zeros((L,), jnp.float32)

        def compute(u):
            usp = jnp.broadcast_to(u, (L,)).astype(jnp.int32)

            @plsc.parallel_loop(0, C)
            def _(i):
                isp = jnp.broadcast_to(i, (L,)).astype(jnp.int32)
                masks = []
                for b in range(NSEQ):
                    ttb = plsc.load_gather(
                        tt4, [usp, jnp.full((L,), b, jnp.int32), isp])
                    masks.append(ttb != 0)
                accs = [zeros] * NSEQ
                sqs = [zeros] * NSEQ
                for j in range(NJ):
                    sl = pl.ds(j * L, L)
                    p = posb[u, i, sl]
                    pt0 = p + type_v[0, sl]
                    pt1 = p + type_v[1, sl]
                    for b in range(NSEQ):
                        e = rows4[u, b, i, sl] + jnp.where(masks[b], pt1, pt0)
                        ebuf[b, i, sl] = e
                        accs[b] = accs[b] + e
                        sqs[b] = sqs[b] + e * e
                for b in range(NSEQ):
                    s1b[b * C + i, ...] = accs[b]
                    s2b[b * C + i, ...] = sqs[b]

            for h in range(NTOK // L):
                s1 = zeros
                s2 = zeros
                for l in range(L):
                    li = jnp.broadcast_to(jnp.int32(l), (L,))
                    s1 = s1 + plsc.load_gather(s1b, [iota + h * L, li])
                    s2 = s2 + plsc.load_gather(s2b, [iota + h * L, li])
                mean = s1 * (1.0 / D)
                var = s2 * (1.0 / D) - mean * mean
                inv = _rsqrt_vec(var + EPS)
                meanb[pl.ds(h * L, L)] = mean
                invb[pl.ds(h * L, L)] = inv

            @plsc.parallel_loop(0, C)
            def _(i):
                isp = jnp.broadcast_to(i, (L,)).astype(jnp.int32)
                ms = []
                vs = []
                for b in range(NSEQ):
                    ms.append(plsc.load_gather(meanb, [isp + (b * C)]))
                    vs.append(plsc.load_gather(invb, [isp + (b * C)]))
                for j in range(NJ):
                    sl = pl.ds(j * L, L)
                    g = gamma_v[sl]
                    bb = beta_v[sl]
                    for b in range(NSEQ):
                        rows4[u, b, i, sl] = ((ebuf[b, i, sl] - ms[b])
                                              * vs[b] * g + bb)

        issue_ids(0, 0)
        issue_pos(0, 0)
        issue_ids(1, 1)
        issue_pos(1, 1)
        wait_ids(0)
        issue_word(0)

        def body(sck, carry):
            u = lax.rem(sck, 2)
            for p in range(2):
                up = p
                u1 = 1 - p

                @pl.when((u == p) & (sck >= 1))
                def _(u1=u1):
                    wait_out(u1)

                @pl.when((u == p) & (sck + 1 < n_sc))
                def _(u1=u1):
                    wait_ids(u1)
                    issue_word(u1)

                @pl.when(u == p)
                def _(up=up):
                    wait_word(up)
                    wait_pos(up)

            compute(u)

            for p in range(2):
                @pl.when(u == p)
                def _(p=p, sck=sck):
                    issue_out(sck, p)

                @pl.when((u == p) & (sck + 2 < n_sc))
                def _(p=p, sck=sck):
                    issue_ids(sck + 2, p)
                    issue_pos(sck + 2, p)
            return carry

        lax.fori_loop(0, n_sc, body, 0)
        wait_out((n_sc - 1) % 2)

    return k(ids, tts, word_emb, pos_emb, type_emb, gamma, beta)


def kernel(input_ids, token_type_ids, attention_mask, word_emb, pos_emb,
           type_emb, gamma, beta):
    b, s = input_ids.shape
    out = _embed_ln(input_ids, token_type_ids,
                    word_emb, pos_emb, type_emb, gamma, beta,
                    n_tokens=b * s, seq_len=s)
    return out, attention_mask


# speedup vs baseline: 4.6360x; 4.6360x over previous
import functools

import jax
import jax.numpy as jnp
from jax import lax
from jax.experimental import pallas as pl
from jax.experimental.pallas import tpu as pltpu
from jax.experimental.pallas import tpu_sc as plsc

D = 768
C = 16
NSLOT = 4
EPS = 1e-12


@functools.partial(jax.jit, static_argnames=("n_tokens",))
def _gather_words(ids, word_emb, *, n_tokens):
    info = plsc.get_sparse_core_info()
    nw = info.num_cores * info.num_subcores
    n_per_w = n_tokens // nw
    n_chunks = n_per_w // C
    mesh = plsc.VectorSubcoreMesh(core_axis_name="c", subcore_axis_name="s")

    scratch = (
        [pltpu.VMEM((C, D), jnp.float32) for _ in range(NSLOT)]
        + [pltpu.VMEM((C,), jnp.int32) for _ in range(NSLOT)]
        + [pltpu.SemaphoreType.DMA for _ in range(3 * NSLOT)]
    )

    @functools.partial(
        pl.kernel,
        out_type=jax.ShapeDtypeStruct((n_tokens, D), jnp.float32),
        mesh=mesh,
        scratch_types=scratch,
    )
    def k(ids_hbm, word_hbm, out_hbm, *sc):
        rows = sc[0:4]
        idsv = sc[4:8]
        sem_ids = sc[8:12]
        sem_w = sc[12:16]
        sem_o = sc[16:20]

        cid = lax.axis_index("c")
        sid = lax.axis_index("s")
        wid = sid * info.num_cores + cid

        def base_of(kk):
            return wid * n_per_w + kk * C

        def issue_ids(kk, slot):
            pltpu.async_copy(ids_hbm.at[pl.ds(base_of(kk), C)], idsv[slot],
                             sem_ids[slot])

        def wait_ids(slot):
            pltpu.make_async_copy(ids_hbm.at[pl.ds(0, C)], idsv[slot],
                                  sem_ids[slot]).wait()

        def issue_word(slot):
            pltpu.async_copy(word_hbm.at[idsv[slot]], rows[slot],
                             sem_w[slot])

        def wait_word(slot):
            pltpu.make_async_copy(word_hbm.at[idsv[slot]], rows[slot],
                                  sem_w[slot]).wait()

        def issue_out(kk, slot):
            pltpu.async_copy(rows[slot], out_hbm.at[pl.ds(base_of(kk), C)],
                             sem_o[slot])

        def wait_out(slot):
            pltpu.make_async_copy(rows[slot], out_hbm.at[pl.ds(0, C)],
                                  sem_o[slot]).wait()

        issue_ids(0, 0)
        issue_ids(1, 1)
        issue_ids(2, 2)
        wait_ids(0)
        issue_word(0)

        def body(k0, carry):
            for p in range(NSLOT):
                kk = k0 * NSLOT + p
                p3 = (p + 3) % NSLOT
                sl1 = (p + 1) % NSLOT
                sl2 = (p + 2) % NSLOT

                @pl.when(kk + 3 < n_chunks)
                def _(p3=p3, kk=kk):
                    issue_ids(kk + 3, p3)

                @pl.when(kk >= 2)
                def _(sl2=sl2):
                    wait_out(sl2)

                @pl.when(kk + 1 < n_chunks)
                def _(sl1=sl1):
                    wait_ids(sl1)
                    issue_word(sl1)

                wait_word(p)
                issue_out(kk, p)
            return carry

        lax.fori_loop(0, n_chunks // NSLOT, body, 0)
        wait_out((n_chunks - 2) % NSLOT)
        wait_out((n_chunks - 1) % NSLOT)

    return k(ids, word_emb)


def _ln_body(stage_ref, tt_ref, pos_ref, type_ref, gamma_ref, beta_ref,
             out_ref):
    e = stage_ref[0] + pos_ref[...]
    tt = tt_ref[0, 0]
    types = jnp.where(tt[:, None] == 0, type_ref[0][None, :],
                      type_ref[1][None, :])
    e = e + types
    mean = jnp.mean(e, axis=-1, keepdims=True)
    var = jnp.mean(e * e, axis=-1, keepdims=True) - mean * mean
    inv = lax.rsqrt(var + EPS)
    out_ref[0] = (e - mean) * inv * gamma_ref[...] + beta_ref[...]


@jax.jit
def _ln(stage, tt3, pos_emb, type_emb, gamma, beta):
    b, s, _ = stage.shape
    return pl.pallas_call(
        _ln_body,
        grid=(b,),
        in_specs=[
            pl.BlockSpec((1, s, D), lambda i: (i, 0, 0)),
            pl.BlockSpec((1, 1, s), lambda i: (i, 0, 0)),
            pl.BlockSpec((s, D), lambda i: (0, 0)),
            pl.BlockSpec(type_emb.shape, lambda i: (0, 0)),
            pl.BlockSpec((D,), lambda i: (0,)),
            pl.BlockSpec((D,), lambda i: (0,)),
        ],
        out_specs=pl.BlockSpec((1, s, D), lambda i: (i, 0, 0)),
        out_shape=jax.ShapeDtypeStruct((b, s, D), jnp.float32),
    )(stage, tt3, pos_emb, type_emb, gamma, beta)


def kernel(input_ids, token_type_ids, attention_mask, word_emb, pos_emb,
           type_emb, gamma, beta):
    b, s = input_ids.shape
    stage = _gather_words(input_ids.reshape(-1), word_emb, n_tokens=b * s)
    out = _ln(stage.reshape(b, s, D), token_type_ids.reshape(b, 1, s),
              pos_emb, type_emb, gamma, beta)
    return out, attention_mask


# speedup vs baseline: 4.8966x; 1.0562x over previous
import functools

import jax
import jax.numpy as jnp
from jax import lax
from jax.experimental import pallas as pl
from jax.experimental.pallas import tpu as pltpu
from jax.experimental.pallas import tpu_sc as plsc

D = 768
C = 16
NSLOT = 4
EPS = 1e-12


@functools.partial(jax.jit, static_argnames=("n_tokens",))
def _gather_words(ids, word_emb, *, n_tokens):
    info = plsc.get_sparse_core_info()
    nw = info.num_cores * info.num_subcores
    n_per_w = n_tokens // nw
    n_chunks = n_per_w // C
    mesh = plsc.VectorSubcoreMesh(core_axis_name="c", subcore_axis_name="s")

    scratch = (
        [pltpu.VMEM((C, D), jnp.float32) for _ in range(NSLOT)]
        + [pltpu.VMEM((C,), jnp.int32) for _ in range(NSLOT)]
        + [pltpu.SemaphoreType.DMA for _ in range(3 * NSLOT)]
    )

    @functools.partial(
        pl.kernel,
        out_type=jax.ShapeDtypeStruct((n_tokens, D), jnp.float32),
        mesh=mesh,
        scratch_types=scratch,
    )
    def k(ids_hbm, word_hbm, out_hbm, *sc):
        rows = sc[0:4]
        idsv = sc[4:8]
        sem_ids = sc[8:12]
        sem_w = sc[12:16]
        sem_o = sc[16:20]

        cid = lax.axis_index("c")
        sid = lax.axis_index("s")
        wid = sid * info.num_cores + cid

        def base_of(kk):
            return wid * n_per_w + kk * C

        def issue_ids(kk, slot):
            pltpu.async_copy(ids_hbm.at[pl.ds(base_of(kk), C)], idsv[slot],
                             sem_ids[slot])

        def wait_ids(slot):
            pltpu.make_async_copy(ids_hbm.at[pl.ds(0, C)], idsv[slot],
                                  sem_ids[slot]).wait()

        def issue_word(slot):
            pltpu.async_copy(word_hbm.at[idsv[slot]], rows[slot],
                             sem_w[slot])

        def wait_word(slot):
            pltpu.make_async_copy(word_hbm.at[idsv[slot]], rows[slot],
                                  sem_w[slot]).wait()

        def issue_out(kk, slot):
            pltpu.async_copy(rows[slot], out_hbm.at[pl.ds(base_of(kk), C)],
                             sem_o[slot])

        def wait_out(slot):
            pltpu.make_async_copy(rows[slot], out_hbm.at[pl.ds(0, C)],
                                  sem_o[slot]).wait()

        issue_ids(0, 0)
        issue_ids(1, 1)
        issue_ids(2, 2)
        wait_ids(0)
        issue_word(0)

        def body(k0, carry):
            for p in range(NSLOT):
                kk = k0 * NSLOT + p
                p3 = (p + 3) % NSLOT
                sl1 = (p + 1) % NSLOT
                sl2 = (p + 2) % NSLOT

                @pl.when(kk + 3 < n_chunks)
                def _(p3=p3, kk=kk):
                    issue_ids(kk + 3, p3)

                @pl.when(kk >= 2)
                def _(sl2=sl2):
                    wait_out(sl2)

                @pl.when(kk + 1 < n_chunks)
                def _(sl1=sl1):
                    wait_ids(sl1)
                    issue_word(sl1)

                wait_word(p)
                issue_out(kk, p)
            return carry

        lax.fori_loop(0, n_chunks // NSLOT, body, 0)
        wait_out((n_chunks - 2) % NSLOT)
        wait_out((n_chunks - 1) % NSLOT)

    return k(ids, word_emb)


def _ln_math(stage_ref, tt_ref, pos_ref, type_ref, gamma_ref, beta_ref,
             out_ref):
    e = stage_ref[0] + pos_ref[...]
    tt = tt_ref[0, 0]
    types = jnp.where(tt[:, None] == 0, type_ref[0][None, :],
                      type_ref[1][None, :])
    e = e + types
    mean = jnp.mean(e, axis=-1, keepdims=True)
    var = jnp.mean(e * e, axis=-1, keepdims=True) - mean * mean
    inv = lax.rsqrt(var + EPS)
    out_ref[0] = (e - mean) * inv * gamma_ref[...] + beta_ref[...]


def _ln_body_first(stage_ref, tt_ref, pos_ref, type_ref, gamma_ref, beta_ref,
                   out_ref):
    _ln_math(stage_ref, tt_ref, pos_ref, type_ref, gamma_ref, beta_ref,
             out_ref)


def _ln_body_second(stage_ref, tt_ref, pos_ref, type_ref, gamma_ref,
                    beta_ref, prev_ref, out_ref):
    del prev_ref
    _ln_math(stage_ref, tt_ref, pos_ref, type_ref, gamma_ref, beta_ref,
             out_ref)


@functools.partial(jax.jit, static_argnames=("total_b", "off"))
def _ln_part(stage, tt3, pos_emb, type_emb, gamma, beta, prev, *,
             total_b, off):
    b, s, _ = stage.shape
    in_specs = [
        pl.BlockSpec((1, s, D), lambda i: (i, 0, 0)),
        pl.BlockSpec((1, 1, s), lambda i: (i, 0, 0)),
        pl.BlockSpec((s, D), lambda i: (0, 0)),
        pl.BlockSpec(type_emb.shape, lambda i: (0, 0)),
        pl.BlockSpec((D,), lambda i: (0,)),
        pl.BlockSpec((D,), lambda i: (0,)),
    ]
    args = [stage, tt3, pos_emb, type_emb, gamma, beta]
    kwargs = {}
    if prev is None:
        body = _ln_body_first
    else:
        body = _ln_body_second
        in_specs.append(pl.BlockSpec(memory_space=pltpu.MemorySpace.HBM))
        args.append(prev)
        kwargs["input_output_aliases"] = {6: 0}
    return pl.pallas_call(
        body,
        grid=(b,),
        in_specs=in_specs,
        out_specs=pl.BlockSpec((1, s, D), lambda i: (i + off, 0, 0)),
        out_shape=jax.ShapeDtypeStruct((total_b, s, D), jnp.float32),
        **kwargs,
    )(*args)


def kernel(input_ids, token_type_ids, attention_mask, word_emb, pos_emb,
           type_emb, gamma, beta):
    b, s = input_ids.shape
    bh = b // 2
    ids_f = input_ids.reshape(-1)
    tt3 = token_type_ids.reshape(b, 1, s)
    st1 = _gather_words(ids_f[: bh * s], word_emb, n_tokens=bh * s)
    st2 = _gather_words(ids_f[bh * s:], word_emb, n_tokens=bh * s)
    o1 = _ln_part(st1.reshape(bh, s, D), tt3[:bh], pos_emb, type_emb,
                  gamma, beta, None, total_b=b, off=0)
    o2 = _ln_part(st2.reshape(bh, s, D), tt3[bh:], pos_emb, type_emb,
                  gamma, beta, o1, total_b=b, off=bh)
    return o2, attention_mask


# speedup vs baseline: 4.9976x; 1.0206x over previous
import functools

import jax
import jax.numpy as jnp
from jax import lax
from jax.experimental import pallas as pl
from jax.experimental.pallas import tpu as pltpu
from jax.experimental.pallas import tpu_sc as plsc

D = 768
C = 16
NSLOT = 4
EPS = 1e-12


@functools.partial(jax.jit, static_argnames=("n_tokens",))
def _gather_words(ids, word_emb, *, n_tokens):
    info = plsc.get_sparse_core_info()
    nw = info.num_cores * info.num_subcores
    n_per_w = n_tokens // nw
    n_chunks = n_per_w // C
    mesh = plsc.VectorSubcoreMesh(core_axis_name="c", subcore_axis_name="s")

    scratch = (
        [pltpu.VMEM((C, D), jnp.float32) for _ in range(NSLOT)]
        + [pltpu.VMEM((C,), jnp.int32) for _ in range(NSLOT)]
        + [pltpu.SemaphoreType.DMA for _ in range(3 * NSLOT)]
    )

    @functools.partial(
        pl.kernel,
        out_type=jax.ShapeDtypeStruct((n_tokens, D), jnp.float32),
        mesh=mesh,
        scratch_types=scratch,
    )
    def k(ids_hbm, word_hbm, out_hbm, *sc):
        rows = sc[0:4]
        idsv = sc[4:8]
        sem_ids = sc[8:12]
        sem_w = sc[12:16]
        sem_o = sc[16:20]

        cid = lax.axis_index("c")
        sid = lax.axis_index("s")
        wid = sid * info.num_cores + cid

        def base_of(kk):
            return wid * n_per_w + kk * C

        def issue_ids(kk, slot):
            pltpu.async_copy(ids_hbm.at[pl.ds(base_of(kk), C)], idsv[slot],
                             sem_ids[slot])

        def wait_ids(slot):
            pltpu.make_async_copy(ids_hbm.at[pl.ds(0, C)], idsv[slot],
                                  sem_ids[slot]).wait()

        def issue_word(slot):
            pltpu.async_copy(word_hbm.at[idsv[slot]], rows[slot],
                             sem_w[slot])

        def wait_word(slot):
            pltpu.make_async_copy(word_hbm.at[idsv[slot]], rows[slot],
                                  sem_w[slot]).wait()

        def issue_out(kk, slot):
            pltpu.async_copy(rows[slot], out_hbm.at[pl.ds(base_of(kk), C)],
                             sem_o[slot])

        def wait_out(slot):
            pltpu.make_async_copy(rows[slot], out_hbm.at[pl.ds(0, C)],
                                  sem_o[slot]).wait()

        issue_ids(0, 0)
        issue_ids(1, 1)
        issue_ids(2, 2)
        wait_ids(0)
        issue_word(0)

        def body(k0, carry):
            for p in range(NSLOT):
                kk = k0 * NSLOT + p
                p3 = (p + 3) % NSLOT
                sl1 = (p + 1) % NSLOT
                sl2 = (p + 2) % NSLOT

                @pl.when(kk + 3 < n_chunks)
                def _(p3=p3, kk=kk):
                    issue_ids(kk + 3, p3)

                @pl.when(kk >= 2)
                def _(sl2=sl2):
                    wait_out(sl2)

                @pl.when(kk + 1 < n_chunks)
                def _(sl1=sl1):
                    wait_ids(sl1)
                    issue_word(sl1)

                wait_word(p)
                issue_out(kk, p)
            return carry

        lax.fori_loop(0, n_chunks // NSLOT, body, 0)
        wait_out((n_chunks - 2) % NSLOT)
        wait_out((n_chunks - 1) % NSLOT)

    return k(ids, word_emb)


def _ln_math(stage_ref, tt_ref, pos_ref, type_ref, gamma_ref, beta_ref,
             out_ref):
    e = stage_ref[0] + pos_ref[...]
    tt = tt_ref[0, 0]
    types = jnp.where(tt[:, None] == 0, type_ref[0][None, :],
                      type_ref[1][None, :])
    e = e + types
    mean = jnp.mean(e, axis=-1, keepdims=True)
    var = jnp.mean(e * e, axis=-1, keepdims=True) - mean * mean
    inv = lax.rsqrt(var + EPS)
    out_ref[0] = (e - mean) * inv * gamma_ref[...] + beta_ref[...]


def _ln_body_first(stage_ref, tt_ref, pos_ref, type_ref, gamma_ref, beta_ref,
                   out_ref):
    _ln_math(stage_ref, tt_ref, pos_ref, type_ref, gamma_ref, beta_ref,
             out_ref)


def _ln_body_second(stage_ref, tt_ref, pos_ref, type_ref, gamma_ref,
                    beta_ref, prev_ref, out_ref):
    del prev_ref
    _ln_math(stage_ref, tt_ref, pos_ref, type_ref, gamma_ref, beta_ref,
             out_ref)


@functools.partial(jax.jit, static_argnames=("total_b", "off"))
def _ln_part(stage, tt3, pos_emb, type_emb, gamma, beta, prev, *,
             total_b, off):
    b, s, _ = stage.shape
    in_specs = [
        pl.BlockSpec((1, s, D), lambda i: (i, 0, 0)),
        pl.BlockSpec((1, 1, s), lambda i: (i, 0, 0)),
        pl.BlockSpec((s, D), lambda i: (0, 0)),
        pl.BlockSpec(type_emb.shape, lambda i: (0, 0)),
        pl.BlockSpec((D,), lambda i: (0,)),
        pl.BlockSpec((D,), lambda i: (0,)),
    ]
    args = [stage, tt3, pos_emb, type_emb, gamma, beta]
    kwargs = {}
    if prev is None:
        body = _ln_body_first
    else:
        body = _ln_body_second
        in_specs.append(pl.BlockSpec(memory_space=pltpu.MemorySpace.HBM))
        args.append(prev)
        kwargs["input_output_aliases"] = {6: 0}
    return pl.pallas_call(
        body,
        grid=(b,),
        in_specs=in_specs,
        out_specs=pl.BlockSpec((1, s, D), lambda i: (i + off, 0, 0)),
        out_shape=jax.ShapeDtypeStruct((total_b, s, D), jnp.float32),
        **kwargs,
    )(*args)


def kernel(input_ids, token_type_ids, attention_mask, word_emb, pos_emb,
           type_emb, gamma, beta):
    b, s = input_ids.shape
    nsplit = 4
    bh = b // nsplit
    ids_f = input_ids.reshape(-1)
    tt3 = token_type_ids.reshape(b, 1, s)
    stages = [
        _gather_words(ids_f[q * bh * s:(q + 1) * bh * s], word_emb,
                      n_tokens=bh * s)
        for q in range(nsplit)
    ]
    out = None
    for q in range(nsplit):
        out = _ln_part(stages[q].reshape(bh, s, D),
                       tt3[q * bh:(q + 1) * bh], pos_emb, type_emb,
                       gamma, beta, out, total_b=b, off=q * bh)
    return out, attention_mask
